# baseline (speedup 1.0000x reference)
import functools

import jax
import jax.numpy as jnp
from jax import lax
from jax.experimental import pallas as pl
from jax.experimental.pallas import tpu as pltpu
from jax.experimental.pallas import tpu_sc as plsc

BATCH = 16384
D = 16
NC = 2
NS = 16
NW = NC * NS
PER_W = BATCH // NW
CHUNK = 128
NCH = PER_W // CHUNK
ROWW = 128
VROWS = (1000000 * D) // ROWW
BPC = CHUNK // D


def _sc_body(gids_hbm, loffs_hbm, umf_hbm, imf_hbm, umlp_hbm, imlp_hbm,
             w_hbm, c_hbm, out_hbm,
             gids_v, loffs_v, umf_v, imf_v, umlp_v, imlp_v, w_v, c_v,
             out_v, sem):
    cid = lax.axis_index("c")
    sid = lax.axis_index("s")
    wid = sid * NC + cid

    pltpu.sync_copy(gids_hbm.at[wid, 0], gids_v)
    pltpu.sync_copy(loffs_hbm.at[wid, 0], loffs_v)
    pltpu.sync_copy(w_hbm, w_v)
    pltpu.sync_copy(c_hbm, c_v)

    iota = lax.iota(jnp.int32, D)
    c_splat = c_v[pl.ds(0, D)]
    wmf_rows = [w_v[0, k, pl.ds(0, D)] for k in range(D)]
    wa_rows = [w_v[1, k, pl.ds(0, D)] for k in range(D)]
    wb_rows = [w_v[2, k, pl.ds(0, D)] for k in range(D)]

    for r in range(NCH):
        copies = [
            pltpu.async_copy(umf_hbm.at[gids_v.at[r]], umf_v, sem),
            pltpu.async_copy(imf_hbm.at[gids_v.at[NCH + r]], imf_v, sem),
            pltpu.async_copy(umlp_hbm.at[gids_v.at[r]], umlp_v, sem),
            pltpu.async_copy(imlp_hbm.at[gids_v.at[NCH + r]], imlp_v, sem),
        ]
        for cp in copies:
            cp.wait()

        def blk(b, _):
            rows = b * D + iota
            ucols = loffs_v[r, pl.ds(b * D, D)]
            icols = loffs_v[NCH + r, pl.ds(b * D, D)]
            acc = c_splat
            for k in range(D):
                u1 = plsc.load_gather(umf_v, [rows, ucols + k])
                i1 = plsc.load_gather(imf_v, [rows, icols + k])
                u2 = plsc.load_gather(umlp_v, [rows, ucols + k])
                i2 = plsc.load_gather(imlp_v, [rows, icols + k])
                acc = (acc + u1 * i1 * wmf_rows[k]
                       + u2 * wa_rows[k] + i2 * wb_rows[k])
            out_v[pl.ds(r * CHUNK + b * D, D)] = 1.0 / (1.0 + jnp.exp(-acc))
            return _

        lax.fori_loop(0, BPC, blk, 0)

    pltpu.sync_copy(out_v, out_hbm.at[pl.ds(wid * PER_W, PER_W)])


@functools.partial(
    pl.kernel,
    out_type=jax.ShapeDtypeStruct((BATCH,), jnp.float32),
    mesh=plsc.VectorSubcoreMesh(core_axis_name="c", subcore_axis_name="s"),
    compiler_params=pltpu.CompilerParams(needs_layout_passes=False),
    scratch_types=[
        pltpu.VMEM((2 * NCH, CHUNK), jnp.int32),
        pltpu.VMEM((2 * NCH, CHUNK), jnp.int32),
        pltpu.VMEM((CHUNK, ROWW), jnp.float32),
        pltpu.VMEM((CHUNK, ROWW), jnp.float32),
        pltpu.VMEM((CHUNK, ROWW), jnp.float32),
        pltpu.VMEM((CHUNK, ROWW), jnp.float32),
        pltpu.VMEM((3, D, 128), jnp.float32),
        pltpu.VMEM((128,), jnp.float32),
        pltpu.VMEM((PER_W,), jnp.float32),
        pltpu.SemaphoreType.DMA,
    ],
)
def _neumf_sc(gids_hbm, loffs_hbm, umf_hbm, imf_hbm, umlp_hbm, imlp_hbm,
              w_hbm, c_hbm, out_hbm, *scratch):
    _sc_body(gids_hbm, loffs_hbm, umf_hbm, imf_hbm, umlp_hbm, imlp_hbm,
             w_hbm, c_hbm, out_hbm, *scratch)


def kernel(X, user_mf, item_mf, user_mlp, item_mlp, W_mlp, b_mlp, W_pred, b_pred):
    Xi = X.astype(jnp.int32)
    gid = Xi >> 3
    loff = (Xi & 7) * D
    pack = lambda A: jnp.concatenate(
        [A[:, 0].reshape(NW, NCH, CHUNK), A[:, 1].reshape(NW, NCH, CHUNK)],
        axis=1).reshape(NW, 1, 2 * NCH, CHUNK)
    gids = pack(gid)
    loffs = pack(loff)

    h = W_pred[D:, 0]
    a = W_mlp[:D, :] @ h
    b = W_mlp[D:, :] @ h
    c = b_mlp @ h + b_pred[0]
    wmf = W_pred[:D, 0]
    w_vecs = jnp.stack([wmf, a, b]).astype(jnp.float32)
    w_rows = jnp.tile(w_vecs[:, :, None], (1, 1, 128))
    c_vec = jnp.full((128,), c, jnp.float32)

    out = _neumf_sc(gids, loffs,
                    user_mf.reshape(VROWS, ROWW), item_mf.reshape(VROWS, ROWW),
                    user_mlp.reshape(VROWS, ROWW), item_mlp.reshape(VROWS, ROWW),
                    w_rows, c_vec)
    return out.reshape(BATCH, 1)
